# baseline (device time: 164020 ns/iter reference)
import functools
import os

import jax
import jax.numpy as jnp
from jax import lax
from jax.experimental import pallas as pl
from jax.experimental.pallas import tpu as pltpu

_COMM = os.environ.get("KERNEL_NO_COMM", "0") != "1"

N_DEV = 4
SQ = 2048
SKV = 2048
HQ_LOCAL = 8
DH = 128
DMODEL = 1024
QBLK = 512
N_QB = SQ // QBLK
CHUNK = SQ // N_DEV
SCALE = 0.08838834764831843
LOCAL_WINDOW = 128
GLOBAL_TOKENS = 32
GBLK = 128
WWIN = 768


def _chunk_rows(c):
    return pl.ds(c * CHUNK, CHUNK)


def _body(x_ref, wq_ref, k_ref, v_ref, wo_ref, out_ref,
          partial_ref, ctx_ref, rs_recv_ref,
          rs_send_sems, rs_recv_sems, ag_send_sems, ag_recv_sems):
    j = pl.program_id(0)
    my = lax.axis_index("i")
    left = lax.rem(my + N_DEV - 1, N_DEV)
    right = lax.rem(my + 1, N_DEV)
    qb = lax.rem(my - j + 2 * N_DEV, N_DEV)
    rows = _chunk_rows(qb)

    q_all = lax.dot_general(
        x_ref[rows, :], wq_ref[...],
        (((1,), (0,)), ((), ())), preferred_element_type=jnp.float32
    ).astype(jnp.bfloat16)

    w0 = jnp.clip(qb * QBLK - LOCAL_WINDOW, GBLK, SKV - WWIN)
    w0 = pl.multiple_of(w0, GBLK)
    qi_g = qb * QBLK + lax.broadcasted_iota(jnp.int32, (QBLK, GBLK), 0)
    ki_g = lax.broadcasted_iota(jnp.int32, (QBLK, GBLK), 1)
    bias_g = jnp.where(
        (ki_g < GLOBAL_TOKENS) | (jnp.abs(qi_g - ki_g) <= LOCAL_WINDOW),
        0.0, -1e9).astype(jnp.float32)
    qi_w = qb * QBLK + lax.broadcasted_iota(jnp.int32, (QBLK, WWIN), 0)
    ki_w = w0 + lax.broadcasted_iota(jnp.int32, (QBLK, WWIN), 1)
    bias_w = jnp.where(jnp.abs(qi_w - ki_w) <= LOCAL_WINDOW,
                       0.0, -1e9).astype(jnp.float32)

    for h in range(HQ_LOCAL):
        hcols = pl.ds(h * DH, DH)
        qh = q_all[:, h * DH:(h + 1) * DH]
        s_g = lax.dot_general(
            qh, k_ref[h, 0:GBLK, :], (((1,), (1,)), ((), ())),
            preferred_element_type=jnp.float32) * SCALE + bias_g
        s_w = lax.dot_general(
            qh, k_ref[h, pl.ds(w0, WWIN), :], (((1,), (1,)), ((), ())),
            preferred_element_type=jnp.float32) * SCALE + bias_w

        m = jnp.maximum(s_g.max(axis=1, keepdims=True),
                        s_w.max(axis=1, keepdims=True))
        e_g = jnp.exp(s_g - m)
        e_w = jnp.exp(s_w - m)
        denom = e_g.sum(axis=1, keepdims=True) + e_w.sum(axis=1, keepdims=True)
        ctx = lax.dot_general(
            (e_g / denom).astype(jnp.bfloat16), v_ref[h, 0:GBLK, :],
            (((1,), (0,)), ((), ())), preferred_element_type=jnp.float32)
        ctx = ctx + lax.dot_general(
            (e_w / denom).astype(jnp.bfloat16), v_ref[h, pl.ds(w0, WWIN), :],
            (((1,), (0,)), ((), ())), preferred_element_type=jnp.float32)
        ctx_ref[:, hcols] = ctx.astype(jnp.bfloat16)

    @pl.when(qb == 0)
    def _():
        for h in range(HQ_LOCAL):
            hcols = pl.ds(h * DH, DH)
            q32 = q_all[0:32, h * DH:(h + 1) * DH]
            s32 = lax.dot_general(
                q32, k_ref[h], (((1,), (1,)), ((), ())),
                preferred_element_type=jnp.float32) * SCALE
            m32 = s32.max(axis=1, keepdims=True)
            e32 = jnp.exp(s32 - m32)
            w32 = (e32 / e32.sum(axis=1, keepdims=True)).astype(jnp.bfloat16)
            ctx32 = lax.dot_general(
                w32, v_ref[h], (((1,), (0,)), ((), ())),
                preferred_element_type=jnp.float32)
            ctx_ref[0:32, hcols] = ctx32.astype(jnp.bfloat16)

    partial_ref[rows, :] = lax.dot_general(
        ctx_ref[...], wo_ref[...], (((1,), (0,)), ((), ())),
        preferred_element_type=jnp.float32)

    def rs_copy(chunk_idx, step):
        return pltpu.make_async_remote_copy(
            src_ref=partial_ref.at[_chunk_rows(chunk_idx)],
            dst_ref=rs_recv_ref.at[step],
            send_sem=rs_send_sems.at[step],
            recv_sem=rs_recv_sems.at[step],
            device_id=(right,),
            device_id_type=pl.DeviceIdType.MESH,
        )

    def rs_send(step):
        rs_copy(lax.rem(my - step + 2 * N_DEV, N_DEV), step).start()

    def rs_wait_and_add(step):
        c = lax.rem(my - step - 1 + 2 * N_DEV, N_DEV)
        rs_copy(c, step).wait_recv()
        rrows = _chunk_rows(c)
        partial_ref[rrows, :] = partial_ref[rrows, :] + rs_recv_ref[step]
        rs_copy(lax.rem(my - step + 2 * N_DEV, N_DEV), step).wait_send()

    def ag_copy(c, sem_idx, target):
        return pltpu.make_async_remote_copy(
            src_ref=out_ref.at[_chunk_rows(c)],
            dst_ref=out_ref.at[_chunk_rows(c)],
            send_sem=ag_send_sems.at[sem_idx],
            recv_sem=ag_recv_sems.at[sem_idx],
            device_id=(target,),
            device_id_type=pl.DeviceIdType.MESH,
        )

    if not _COMM:
        @pl.when(j == N_QB - 1)
        def _():
            out_ref[...] = partial_ref[...]
        return

    @pl.when(j == 0)
    def _():
        barrier_sem = pltpu.get_barrier_semaphore()
        for nbr in (left, right):
            pl.semaphore_signal(barrier_sem, inc=1, device_id=(nbr,),
                                device_id_type=pl.DeviceIdType.MESH)
        pl.semaphore_wait(barrier_sem, 2)
        rs_send(0)

    for step in range(N_QB - 2):
        @pl.when(j == step + 1)
        def _():
            rs_wait_and_add(step)
            rs_send(step + 1)

    @pl.when(j == N_QB - 1)
    def _():
        rs_wait_and_add(N_DEV - 2)

        c_own = lax.rem(my + 1, N_DEV)
        out_ref[_chunk_rows(c_own), :] = partial_ref[_chunk_rows(c_own), :]

        send_r = ag_copy(c_own, 0, right)
        send_l = ag_copy(c_own, 1, left)
        send_r.start()
        send_l.start()

        ag_copy(my, 0, right).wait_recv()
        fwd = ag_copy(my, 2, right)
        fwd.start()

        ag_copy(lax.rem(my + 2, N_DEV), 1, right).wait_recv()
        ag_copy(left, 2, right).wait_recv()

        send_r.wait_send()
        send_l.wait_send()
        fwd.wait_send()

        @functools.partial(pl.run_scoped,
                           second_barrier=pltpu.SemaphoreType.REGULAR)
        def _(second_barrier):
            for nbr in (left, right):
                pl.semaphore_signal(second_barrier, inc=1, device_id=(nbr,),
                                    device_id_type=pl.DeviceIdType.MESH)
            pl.semaphore_wait(second_barrier, 2)


def kernel(x, Wq, K_ext, V_ext, Wo):
    my = lax.axis_index("i")
    x2 = x.reshape(SQ, DMODEL).astype(jnp.bfloat16)
    wq_s = lax.dynamic_slice(
        Wq, (0, my * HQ_LOCAL * DH), (DMODEL, HQ_LOCAL * DH)
    ).astype(jnp.bfloat16)
    wo_s = lax.dynamic_slice(
        Wo, (my * HQ_LOCAL * DH, 0), (HQ_LOCAL * DH, DMODEL)
    ).astype(jnp.bfloat16)
    k = jnp.transpose(K_ext.reshape(SKV, HQ_LOCAL, DH), (1, 0, 2)).astype(
        jnp.bfloat16)
    v = jnp.transpose(V_ext.reshape(SKV, HQ_LOCAL, DH), (1, 0, 2)).astype(
        jnp.bfloat16)

    out = pl.pallas_call(
        _body,
        grid=(N_QB,),
        in_specs=[
            pl.BlockSpec((SQ, DMODEL), lambda j: (0, 0)),
            pl.BlockSpec((DMODEL, HQ_LOCAL * DH), lambda j: (0, 0)),
            pl.BlockSpec((HQ_LOCAL, SKV, DH), lambda j: (0, 0, 0)),
            pl.BlockSpec((HQ_LOCAL, SKV, DH), lambda j: (0, 0, 0)),
            pl.BlockSpec((HQ_LOCAL * DH, DMODEL), lambda j: (0, 0)),
        ],
        out_specs=pl.BlockSpec((SQ, DMODEL), lambda j: (0, 0)),
        out_shape=jax.ShapeDtypeStruct((SQ, DMODEL), jnp.float32),
        scratch_shapes=[
            pltpu.VMEM((SQ, DMODEL), jnp.float32),
            pltpu.VMEM((QBLK, HQ_LOCAL * DH), jnp.bfloat16),
            pltpu.VMEM((N_DEV - 1, CHUNK, DMODEL), jnp.float32),
            pltpu.SemaphoreType.DMA((N_DEV - 1,)),
            pltpu.SemaphoreType.DMA((N_DEV - 1,)),
            pltpu.SemaphoreType.DMA((N_DEV - 1,)),
            pltpu.SemaphoreType.DMA((N_DEV - 1,)),
        ],
        compiler_params=pltpu.CompilerParams(
            collective_id=0,
            dimension_semantics=("arbitrary",),
        ),
    )(x2, wq_s, k, v, wo_s)
    return out.reshape(1, SQ, DMODEL)


# device time: 141563 ns/iter; 1.1586x vs baseline; 1.1586x over previous
import functools
import os

import jax
import jax.numpy as jnp
from jax import lax
from jax.experimental import pallas as pl
from jax.experimental.pallas import tpu as pltpu

_COMM = os.environ.get("KERNEL_NO_COMM", "0") != "1"

N_DEV = 4
SQ = 2048
SKV = 2048
HQ_LOCAL = 8
DH = 128
DMODEL = 1024
QBLK = 512
N_QB = SQ // QBLK
CHUNK = SQ // N_DEV
SCALE = 0.08838834764831843
LOCAL_WINDOW = 128
GLOBAL_TOKENS = 32
GBLK = 128
WWIN = 768


def _chunk_rows(c):
    return pl.ds(c * CHUNK, CHUNK)


def _body(x_ref, wq_ref, k_ref, v_ref, wo_ref, out_ref,
          partial_ref, ctx_ref, rs_recv_ref, ag_buf,
          rs_send_sems, rs_recv_sems, ag_send_sems, ag_recv_sems):
    j = pl.program_id(0)
    my = lax.axis_index("i")
    left = lax.rem(my + N_DEV - 1, N_DEV)
    right = lax.rem(my + 1, N_DEV)
    qb = lax.rem(my - j + 2 * N_DEV, N_DEV)
    rows = _chunk_rows(qb)

    q_all = lax.dot_general(
        x_ref[rows, :], wq_ref[...],
        (((1,), (0,)), ((), ())), preferred_element_type=jnp.float32
    ).astype(jnp.bfloat16)

    w0 = jnp.clip(qb * QBLK - LOCAL_WINDOW, GBLK, SKV - WWIN)
    w0 = pl.multiple_of(w0, GBLK)
    qi_g = qb * QBLK + lax.broadcasted_iota(jnp.int32, (QBLK, GBLK), 0)
    ki_g = lax.broadcasted_iota(jnp.int32, (QBLK, GBLK), 1)
    bias_g = jnp.where(
        (ki_g < GLOBAL_TOKENS) | (jnp.abs(qi_g - ki_g) <= LOCAL_WINDOW),
        0.0, -1e9).astype(jnp.float32)
    qi_w = qb * QBLK + lax.broadcasted_iota(jnp.int32, (QBLK, WWIN), 0)
    ki_w = w0 + lax.broadcasted_iota(jnp.int32, (QBLK, WWIN), 1)
    bias_w = jnp.where(jnp.abs(qi_w - ki_w) <= LOCAL_WINDOW,
                       0.0, -1e9).astype(jnp.float32)

    for h in range(HQ_LOCAL):
        hcols = pl.ds(h * DH, DH)
        qh = q_all[:, h * DH:(h + 1) * DH]
        s_g = lax.dot_general(
            qh, k_ref[h, 0:GBLK, :], (((1,), (1,)), ((), ())),
            preferred_element_type=jnp.float32) * SCALE + bias_g
        s_w = lax.dot_general(
            qh, k_ref[h, pl.ds(w0, WWIN), :], (((1,), (1,)), ((), ())),
            preferred_element_type=jnp.float32) * SCALE + bias_w

        m = jnp.maximum(s_g.max(axis=1, keepdims=True),
                        s_w.max(axis=1, keepdims=True))
        e_g = jnp.exp(s_g - m)
        e_w = jnp.exp(s_w - m)
        denom = e_g.sum(axis=1, keepdims=True) + e_w.sum(axis=1, keepdims=True)
        ctx = lax.dot_general(
            (e_g / denom).astype(jnp.bfloat16), v_ref[h, 0:GBLK, :],
            (((1,), (0,)), ((), ())), preferred_element_type=jnp.float32)
        ctx = ctx + lax.dot_general(
            (e_w / denom).astype(jnp.bfloat16), v_ref[h, pl.ds(w0, WWIN), :],
            (((1,), (0,)), ((), ())), preferred_element_type=jnp.float32)
        ctx_ref[:, hcols] = ctx.astype(jnp.bfloat16)

    @pl.when(qb == 0)
    def _():
        for h in range(HQ_LOCAL):
            hcols = pl.ds(h * DH, DH)
            q32 = q_all[0:32, h * DH:(h + 1) * DH]
            s32 = lax.dot_general(
                q32, k_ref[h], (((1,), (1,)), ((), ())),
                preferred_element_type=jnp.float32) * SCALE
            m32 = s32.max(axis=1, keepdims=True)
            e32 = jnp.exp(s32 - m32)
            w32 = (e32 / e32.sum(axis=1, keepdims=True)).astype(jnp.bfloat16)
            ctx32 = lax.dot_general(
                w32, v_ref[h], (((1,), (0,)), ((), ())),
                preferred_element_type=jnp.float32)
            ctx_ref[0:32, hcols] = ctx32.astype(jnp.bfloat16)

    partial_ref[rows, :] = lax.dot_general(
        ctx_ref[...], wo_ref[...], (((1,), (0,)), ((), ())),
        preferred_element_type=jnp.float32)

    def rs_copy(chunk_idx, step):
        return pltpu.make_async_remote_copy(
            src_ref=partial_ref.at[_chunk_rows(chunk_idx)],
            dst_ref=rs_recv_ref.at[step],
            send_sem=rs_send_sems.at[step],
            recv_sem=rs_recv_sems.at[step],
            device_id=(right,),
            device_id_type=pl.DeviceIdType.MESH,
        )

    def rs_send(step):
        rs_copy(lax.rem(my - step + 2 * N_DEV, N_DEV), step).start()

    def rs_wait_and_add(step):
        c = lax.rem(my - step - 1 + 2 * N_DEV, N_DEV)
        rs_copy(c, step).wait_recv()
        rrows = _chunk_rows(c)
        partial_ref[rrows, :] = partial_ref[rrows, :] + rs_recv_ref[step]
        rs_copy(lax.rem(my - step + 2 * N_DEV, N_DEV), step).wait_send()

    def ag_copy(c, sem_idx, target):
        return pltpu.make_async_remote_copy(
            src_ref=ag_buf.at[c],
            dst_ref=ag_buf.at[c],
            send_sem=ag_send_sems.at[sem_idx],
            recv_sem=ag_recv_sems.at[sem_idx],
            device_id=(target,),
            device_id_type=pl.DeviceIdType.MESH,
        )

    if not _COMM:
        @pl.when(j == N_QB - 1)
        def _():
            out_ref[...] = partial_ref[...]
        return

    @pl.when(j == 0)
    def _():
        barrier_sem = pltpu.get_barrier_semaphore()
        for nbr in (left, right):
            pl.semaphore_signal(barrier_sem, inc=1, device_id=(nbr,),
                                device_id_type=pl.DeviceIdType.MESH)
        pl.semaphore_wait(barrier_sem, 2)
        rs_send(0)

    for step in range(N_QB - 2):
        @pl.when(j == step + 1)
        def _():
            rs_wait_and_add(step)
            rs_send(step + 1)

    @pl.when(j == N_QB - 1)
    def _():
        rs_wait_and_add(N_DEV - 2)

        c_own = lax.rem(my + 1, N_DEV)
        orows = _chunk_rows(c_own)
        ag_buf[c_own] = partial_ref[orows, :].astype(jnp.bfloat16)

        send_r = ag_copy(c_own, 0, right)
        send_l = ag_copy(c_own, 1, left)
        send_r.start()
        send_l.start()

        out_ref[orows, :] = partial_ref[orows, :]

        ag_copy(my, 0, right).wait_recv()
        fwd = ag_copy(my, 2, right)
        fwd.start()
        out_ref[_chunk_rows(my), :] = ag_buf[my].astype(jnp.float32)

        c_r = lax.rem(my + 2, N_DEV)
        ag_copy(c_r, 1, right).wait_recv()
        out_ref[_chunk_rows(c_r), :] = ag_buf[c_r].astype(jnp.float32)
        ag_copy(left, 2, right).wait_recv()
        out_ref[_chunk_rows(left), :] = ag_buf[left].astype(jnp.float32)

        send_r.wait_send()
        send_l.wait_send()
        fwd.wait_send()

        @functools.partial(pl.run_scoped,
                           second_barrier=pltpu.SemaphoreType.REGULAR)
        def _(second_barrier):
            for nbr in (left, right):
                pl.semaphore_signal(second_barrier, inc=1, device_id=(nbr,),
                                    device_id_type=pl.DeviceIdType.MESH)
            pl.semaphore_wait(second_barrier, 2)


def kernel(x, Wq, K_ext, V_ext, Wo):
    my = lax.axis_index("i")
    x2 = x.reshape(SQ, DMODEL).astype(jnp.bfloat16)
    wq_s = lax.dynamic_slice(
        Wq, (0, my * HQ_LOCAL * DH), (DMODEL, HQ_LOCAL * DH)
    ).astype(jnp.bfloat16)
    wo_s = lax.dynamic_slice(
        Wo, (my * HQ_LOCAL * DH, 0), (HQ_LOCAL * DH, DMODEL)
    ).astype(jnp.bfloat16)
    k = jnp.transpose(K_ext.reshape(SKV, HQ_LOCAL, DH), (1, 0, 2)).astype(
        jnp.bfloat16)
    v = jnp.transpose(V_ext.reshape(SKV, HQ_LOCAL, DH), (1, 0, 2)).astype(
        jnp.bfloat16)

    out = pl.pallas_call(
        _body,
        grid=(N_QB,),
        in_specs=[
            pl.BlockSpec((SQ, DMODEL), lambda j: (0, 0)),
            pl.BlockSpec((DMODEL, HQ_LOCAL * DH), lambda j: (0, 0)),
            pl.BlockSpec((HQ_LOCAL, SKV, DH), lambda j: (0, 0, 0)),
            pl.BlockSpec((HQ_LOCAL, SKV, DH), lambda j: (0, 0, 0)),
            pl.BlockSpec((HQ_LOCAL * DH, DMODEL), lambda j: (0, 0)),
        ],
        out_specs=pl.BlockSpec((SQ, DMODEL), lambda j: (0, 0)),
        out_shape=jax.ShapeDtypeStruct((SQ, DMODEL), jnp.float32),
        scratch_shapes=[
            pltpu.VMEM((SQ, DMODEL), jnp.float32),
            pltpu.VMEM((QBLK, HQ_LOCAL * DH), jnp.bfloat16),
            pltpu.VMEM((N_DEV - 1, CHUNK, DMODEL), jnp.float32),
            pltpu.VMEM((N_DEV, CHUNK, DMODEL), jnp.bfloat16),
            pltpu.SemaphoreType.DMA((N_DEV - 1,)),
            pltpu.SemaphoreType.DMA((N_DEV - 1,)),
            pltpu.SemaphoreType.DMA((N_DEV - 1,)),
            pltpu.SemaphoreType.DMA((N_DEV - 1,)),
        ],
        compiler_params=pltpu.CompilerParams(
            dimension_semantics=("arbitrary",),
            **({"collective_id": 0} if _COMM else {}),
        ),
    )(x2, wq_s, k, v, wo_s)
    return out.reshape(1, SQ, DMODEL)


# device time: 111320 ns/iter; 1.4734x vs baseline; 1.2717x over previous
import functools
import os

import jax
import jax.numpy as jnp
from jax import lax
from jax.experimental import pallas as pl
from jax.experimental.pallas import tpu as pltpu

_COMM = os.environ.get("KERNEL_NO_COMM", "0") != "1"

N_DEV = 4
SQ = 2048
SKV = 2048
HQ_LOCAL = 8
DH = 128
DMODEL = 1024
QBLK = 512
N_QB = SQ // QBLK
CHUNK = SQ // N_DEV
SCALE = 0.08838834764831843
LOCAL_WINDOW = 128
GLOBAL_TOKENS = 32
GBLK = 128
WWIN = 768


def _chunk_rows(c):
    return pl.ds(c * CHUNK, CHUNK)


def _body(x_ref, wq_ref, k_ref, v_ref, wo_ref, out_ref,
          partial_ref, ctx_ref, rs_send_buf, rs_recv_ref, ag_buf,
          rs_send_sems, rs_recv_sems, ag_send_sems, ag_recv_sems):
    j = pl.program_id(0)
    my = lax.axis_index("i")
    left = lax.rem(my + N_DEV - 1, N_DEV)
    right = lax.rem(my + 1, N_DEV)
    qb = lax.rem(my - j + 2 * N_DEV, N_DEV)
    rows = _chunk_rows(qb)

    q_all = lax.dot_general(
        x_ref[rows, :], wq_ref[...],
        (((1,), (0,)), ((), ())), preferred_element_type=jnp.float32
    ).astype(jnp.bfloat16)

    w0 = jnp.clip(qb * QBLK - LOCAL_WINDOW, GBLK, SKV - WWIN)
    w0 = pl.multiple_of(w0, GBLK)
    qi_g = qb * QBLK + lax.broadcasted_iota(jnp.int32, (QBLK, GBLK), 0)
    ki_g = lax.broadcasted_iota(jnp.int32, (QBLK, GBLK), 1)
    bias_g = jnp.where(
        (ki_g < GLOBAL_TOKENS) | (jnp.abs(qi_g - ki_g) <= LOCAL_WINDOW),
        0.0, -1e9).astype(jnp.float32)
    qi_w = qb * QBLK + lax.broadcasted_iota(jnp.int32, (QBLK, WWIN), 0)
    ki_w = w0 + lax.broadcasted_iota(jnp.int32, (QBLK, WWIN), 1)
    bias_w = jnp.where(jnp.abs(qi_w - ki_w) <= LOCAL_WINDOW,
                       0.0, -1e9).astype(jnp.float32)

    for h in range(HQ_LOCAL):
        hcols = pl.ds(h * DH, DH)
        qh = q_all[:, h * DH:(h + 1) * DH]
        s_g = lax.dot_general(
            qh, k_ref[h, 0:GBLK, :], (((1,), (1,)), ((), ())),
            preferred_element_type=jnp.float32) * SCALE + bias_g
        s_w = lax.dot_general(
            qh, k_ref[h, pl.ds(w0, WWIN), :], (((1,), (1,)), ((), ())),
            preferred_element_type=jnp.float32) * SCALE + bias_w

        m = jnp.maximum(s_g.max(axis=1, keepdims=True),
                        s_w.max(axis=1, keepdims=True))
        e_g = jnp.exp(s_g - m)
        e_w = jnp.exp(s_w - m)
        denom = e_g.sum(axis=1, keepdims=True) + e_w.sum(axis=1, keepdims=True)
        ctx = lax.dot_general(
            (e_g / denom).astype(jnp.bfloat16), v_ref[h, 0:GBLK, :],
            (((1,), (0,)), ((), ())), preferred_element_type=jnp.float32)
        ctx = ctx + lax.dot_general(
            (e_w / denom).astype(jnp.bfloat16), v_ref[h, pl.ds(w0, WWIN), :],
            (((1,), (0,)), ((), ())), preferred_element_type=jnp.float32)
        ctx_ref[:, hcols] = ctx.astype(jnp.bfloat16)

    @pl.when(qb == 0)
    def _():
        for h in range(HQ_LOCAL):
            hcols = pl.ds(h * DH, DH)
            q32 = q_all[0:32, h * DH:(h + 1) * DH]
            s32 = lax.dot_general(
                q32, k_ref[h], (((1,), (1,)), ((), ())),
                preferred_element_type=jnp.float32) * SCALE
            m32 = s32.max(axis=1, keepdims=True)
            e32 = jnp.exp(s32 - m32)
            w32 = (e32 / e32.sum(axis=1, keepdims=True)).astype(jnp.bfloat16)
            ctx32 = lax.dot_general(
                w32, v_ref[h], (((1,), (0,)), ((), ())),
                preferred_element_type=jnp.float32)
            ctx_ref[0:32, hcols] = ctx32.astype(jnp.bfloat16)

    partial_ref[rows, :] = lax.dot_general(
        ctx_ref[...], wo_ref[...], (((1,), (0,)), ((), ())),
        preferred_element_type=jnp.float32)

    def rs_copy(step):
        return pltpu.make_async_remote_copy(
            src_ref=rs_send_buf.at[step],
            dst_ref=rs_recv_ref.at[step],
            send_sem=rs_send_sems.at[step],
            recv_sem=rs_recv_sems.at[step],
            device_id=(right,),
            device_id_type=pl.DeviceIdType.MESH,
        )

    def rs_send(step):
        c = lax.rem(my - step + 2 * N_DEV, N_DEV)
        rs_send_buf[step] = partial_ref[_chunk_rows(c), :].astype(jnp.bfloat16)
        rs_copy(step).start()

    def rs_wait_and_add(step):
        c = lax.rem(my - step - 1 + 2 * N_DEV, N_DEV)
        rs_copy(step).wait_recv()
        rrows = _chunk_rows(c)
        partial_ref[rrows, :] = partial_ref[rrows, :] + rs_recv_ref[step]
        rs_copy(step).wait_send()

    def ag_copy(c, sem_idx, target):
        return pltpu.make_async_remote_copy(
            src_ref=ag_buf.at[c],
            dst_ref=ag_buf.at[c],
            send_sem=ag_send_sems.at[sem_idx],
            recv_sem=ag_recv_sems.at[sem_idx],
            device_id=(target,),
            device_id_type=pl.DeviceIdType.MESH,
        )

    if not _COMM:
        @pl.when(j == N_QB - 1)
        def _():
            out_ref[...] = partial_ref[...]
        return

    @pl.when(j == 0)
    def _():
        barrier_sem = pltpu.get_barrier_semaphore()
        for nbr in (left, right):
            pl.semaphore_signal(barrier_sem, inc=1, device_id=(nbr,),
                                device_id_type=pl.DeviceIdType.MESH)
        pl.semaphore_wait(barrier_sem, 2)
        rs_send(0)

    for step in range(N_QB - 2):
        @pl.when(j == step + 1)
        def _():
            rs_wait_and_add(step)
            rs_send(step + 1)

    @pl.when(j == N_QB - 1)
    def _():
        rs_wait_and_add(N_DEV - 2)

        c_own = lax.rem(my + 1, N_DEV)
        orows = _chunk_rows(c_own)
        ag_buf[c_own] = partial_ref[orows, :].astype(jnp.bfloat16)

        send_r = ag_copy(c_own, 0, right)
        send_l = ag_copy(c_own, 1, left)
        send_r.start()
        send_l.start()

        out_ref[orows, :] = partial_ref[orows, :]

        ag_copy(my, 0, right).wait_recv()
        fwd = ag_copy(my, 2, right)
        fwd.start()
        out_ref[_chunk_rows(my), :] = ag_buf[my].astype(jnp.float32)

        c_r = lax.rem(my + 2, N_DEV)
        ag_copy(c_r, 1, right).wait_recv()
        out_ref[_chunk_rows(c_r), :] = ag_buf[c_r].astype(jnp.float32)
        ag_copy(left, 2, right).wait_recv()
        out_ref[_chunk_rows(left), :] = ag_buf[left].astype(jnp.float32)

        send_r.wait_send()
        send_l.wait_send()
        fwd.wait_send()

        @functools.partial(pl.run_scoped,
                           second_barrier=pltpu.SemaphoreType.REGULAR)
        def _(second_barrier):
            for nbr in (left, right):
                pl.semaphore_signal(second_barrier, inc=1, device_id=(nbr,),
                                    device_id_type=pl.DeviceIdType.MESH)
            pl.semaphore_wait(second_barrier, 2)


def kernel(x, Wq, K_ext, V_ext, Wo):
    my = lax.axis_index("i")
    x2 = x.reshape(SQ, DMODEL).astype(jnp.bfloat16)
    wq_s = lax.dynamic_slice(
        Wq, (0, my * HQ_LOCAL * DH), (DMODEL, HQ_LOCAL * DH)
    ).astype(jnp.bfloat16)
    wo_s = lax.dynamic_slice(
        Wo, (my * HQ_LOCAL * DH, 0), (HQ_LOCAL * DH, DMODEL)
    ).astype(jnp.bfloat16)
    k = jnp.transpose(K_ext.reshape(SKV, HQ_LOCAL, DH), (1, 0, 2)).astype(
        jnp.bfloat16)
    v = jnp.transpose(V_ext.reshape(SKV, HQ_LOCAL, DH), (1, 0, 2)).astype(
        jnp.bfloat16)

    out = pl.pallas_call(
        _body,
        grid=(N_QB,),
        in_specs=[
            pl.BlockSpec((SQ, DMODEL), lambda j: (0, 0)),
            pl.BlockSpec((DMODEL, HQ_LOCAL * DH), lambda j: (0, 0)),
            pl.BlockSpec((HQ_LOCAL, SKV, DH), lambda j: (0, 0, 0)),
            pl.BlockSpec((HQ_LOCAL, SKV, DH), lambda j: (0, 0, 0)),
            pl.BlockSpec((HQ_LOCAL * DH, DMODEL), lambda j: (0, 0)),
        ],
        out_specs=pl.BlockSpec((SQ, DMODEL), lambda j: (0, 0)),
        out_shape=jax.ShapeDtypeStruct((SQ, DMODEL), jnp.float32),
        scratch_shapes=[
            pltpu.VMEM((SQ, DMODEL), jnp.float32),
            pltpu.VMEM((QBLK, HQ_LOCAL * DH), jnp.bfloat16),
            pltpu.VMEM((N_DEV - 1, CHUNK, DMODEL), jnp.bfloat16),
            pltpu.VMEM((N_DEV - 1, CHUNK, DMODEL), jnp.bfloat16),
            pltpu.VMEM((N_DEV, CHUNK, DMODEL), jnp.bfloat16),
            pltpu.SemaphoreType.DMA((N_DEV - 1,)),
            pltpu.SemaphoreType.DMA((N_DEV - 1,)),
            pltpu.SemaphoreType.DMA((N_DEV - 1,)),
            pltpu.SemaphoreType.DMA((N_DEV - 1,)),
        ],
        compiler_params=pltpu.CompilerParams(
            dimension_semantics=("arbitrary",),
            **({"collective_id": 0} if _COMM else {}),
        ),
    )(x2, wq_s, k, v, wo_s)
    return out.reshape(1, SQ, DMODEL)


# device time: 104938 ns/iter; 1.5630x vs baseline; 1.0608x over previous
import functools
import os

import jax
import jax.numpy as jnp
from jax import lax
from jax.experimental import pallas as pl
from jax.experimental.pallas import tpu as pltpu

_COMM = os.environ.get("KERNEL_NO_COMM", "0") != "1"

N_DEV = 4
SQ = 2048
SKV = 2048
HQ_LOCAL = 8
DH = 128
DMODEL = 1024
QBLK = 512
N_QB = SQ // QBLK
CHUNK = SQ // N_DEV
SCALE = 0.08838834764831843
LOCAL_WINDOW = 128
GLOBAL_TOKENS = 32
GBLK = 32
WWIN = 768


def _chunk_rows(c):
    return pl.ds(c * CHUNK, CHUNK)


def _body(x_ref, wq_ref, k_ref, v_ref, wo_ref, out_ref,
          partial_ref, ctx_ref, rs_send_buf, rs_recv_ref, ag_buf,
          rs_send_sems, rs_recv_sems, ag_send_sems, ag_recv_sems):
    j = pl.program_id(0)
    my = lax.axis_index("i")
    left = lax.rem(my + N_DEV - 1, N_DEV)
    right = lax.rem(my + 1, N_DEV)
    qb = lax.rem(my - j + 2 * N_DEV, N_DEV)
    rows = _chunk_rows(qb)

    q_all = (lax.dot_general(
        x_ref[rows, :], wq_ref[...],
        (((1,), (0,)), ((), ())), preferred_element_type=jnp.float32
    ) * SCALE).astype(jnp.bfloat16)

    w0 = jnp.clip(qb * QBLK - LOCAL_WINDOW, 0, SKV - WWIN)
    w0 = pl.multiple_of(w0, 128)
    qi_w = qb * QBLK + lax.broadcasted_iota(jnp.int32, (QBLK, WWIN), 0)
    ki_w = w0 + lax.broadcasted_iota(jnp.int32, (QBLK, WWIN), 1)
    bias_w = jnp.where(
        (jnp.abs(qi_w - ki_w) <= LOCAL_WINDOW) | (ki_w < GLOBAL_TOKENS),
        0.0, -1e9).astype(jnp.float32)
    bias_g = jnp.where(qb == 0, -1e9, 0.0).astype(jnp.float32)

    for h in range(HQ_LOCAL):
        hcols = pl.ds(h * DH, DH)
        qh = q_all[:, h * DH:(h + 1) * DH]
        s_g = lax.dot_general(
            qh, k_ref[h, 0:GBLK, :], (((1,), (1,)), ((), ())),
            preferred_element_type=jnp.float32) + bias_g
        s_w = lax.dot_general(
            qh, k_ref[h, pl.ds(w0, WWIN), :], (((1,), (1,)), ((), ())),
            preferred_element_type=jnp.float32) + bias_w

        e_g = jnp.exp(s_g)
        e_w = jnp.exp(s_w)
        inv = 1.0 / (e_g.sum(axis=1, keepdims=True)
                     + e_w.sum(axis=1, keepdims=True))
        ctx = lax.dot_general(
            (e_g * inv).astype(jnp.bfloat16), v_ref[h, 0:GBLK, :],
            (((1,), (0,)), ((), ())), preferred_element_type=jnp.float32)
        ctx = ctx + lax.dot_general(
            (e_w * inv).astype(jnp.bfloat16), v_ref[h, pl.ds(w0, WWIN), :],
            (((1,), (0,)), ((), ())), preferred_element_type=jnp.float32)
        ctx_ref[:, hcols] = ctx.astype(jnp.bfloat16)

    @pl.when(qb == 0)
    def _():
        for h in range(HQ_LOCAL):
            hcols = pl.ds(h * DH, DH)
            q32 = q_all[0:32, h * DH:(h + 1) * DH]
            s32 = lax.dot_general(
                q32, k_ref[h], (((1,), (1,)), ((), ())),
                preferred_element_type=jnp.float32)
            e32 = jnp.exp(s32)
            w32 = (e32 / e32.sum(axis=1, keepdims=True)).astype(jnp.bfloat16)
            ctx32 = lax.dot_general(
                w32, v_ref[h], (((1,), (0,)), ((), ())),
                preferred_element_type=jnp.float32)
            ctx_ref[0:32, hcols] = ctx32.astype(jnp.bfloat16)

    partial_ref[rows, :] = lax.dot_general(
        ctx_ref[...], wo_ref[...], (((1,), (0,)), ((), ())),
        preferred_element_type=jnp.float32)

    def rs_copy(step):
        return pltpu.make_async_remote_copy(
            src_ref=rs_send_buf.at[step],
            dst_ref=rs_recv_ref.at[step],
            send_sem=rs_send_sems.at[step],
            recv_sem=rs_recv_sems.at[step],
            device_id=(right,),
            device_id_type=pl.DeviceIdType.MESH,
        )

    def rs_send(step):
        c = lax.rem(my - step + 2 * N_DEV, N_DEV)
        rs_send_buf[step] = partial_ref[_chunk_rows(c), :].astype(jnp.bfloat16)
        rs_copy(step).start()

    def rs_wait_and_add(step):
        c = lax.rem(my - step - 1 + 2 * N_DEV, N_DEV)
        rs_copy(step).wait_recv()
        rrows = _chunk_rows(c)
        partial_ref[rrows, :] = partial_ref[rrows, :] + rs_recv_ref[step]
        rs_copy(step).wait_send()

    def ag_copy(c, sem_idx, target):
        return pltpu.make_async_remote_copy(
            src_ref=ag_buf.at[c],
            dst_ref=ag_buf.at[c],
            send_sem=ag_send_sems.at[sem_idx],
            recv_sem=ag_recv_sems.at[sem_idx],
            device_id=(target,),
            device_id_type=pl.DeviceIdType.MESH,
        )

    if not _COMM:
        @pl.when(j == N_QB - 1)
        def _():
            out_ref[...] = partial_ref[...]
        return

    @pl.when(j == 0)
    def _():
        barrier_sem = pltpu.get_barrier_semaphore()
        for nbr in (left, right):
            pl.semaphore_signal(barrier_sem, inc=1, device_id=(nbr,),
                                device_id_type=pl.DeviceIdType.MESH)
        pl.semaphore_wait(barrier_sem, 2)
        rs_send(0)

    for step in range(N_QB - 2):
        @pl.when(j == step + 1)
        def _():
            rs_wait_and_add(step)
            rs_send(step + 1)

    @pl.when(j == N_QB - 1)
    def _():
        rs_wait_and_add(N_DEV - 2)

        c_own = lax.rem(my + 1, N_DEV)
        orows = _chunk_rows(c_own)
        ag_buf[c_own] = partial_ref[orows, :].astype(jnp.bfloat16)

        send_r = ag_copy(c_own, 0, right)
        send_l = ag_copy(c_own, 1, left)
        send_r.start()
        send_l.start()

        out_ref[orows, :] = partial_ref[orows, :]

        ag_copy(my, 0, right).wait_recv()
        fwd = ag_copy(my, 2, right)
        fwd.start()
        out_ref[_chunk_rows(my), :] = ag_buf[my].astype(jnp.float32)

        c_r = lax.rem(my + 2, N_DEV)
        ag_copy(c_r, 1, right).wait_recv()
        out_ref[_chunk_rows(c_r), :] = ag_buf[c_r].astype(jnp.float32)
        ag_copy(left, 2, right).wait_recv()
        out_ref[_chunk_rows(left), :] = ag_buf[left].astype(jnp.float32)

        send_r.wait_send()
        send_l.wait_send()
        fwd.wait_send()

        @functools.partial(pl.run_scoped,
                           second_barrier=pltpu.SemaphoreType.REGULAR)
        def _(second_barrier):
            for nbr in (left, right):
                pl.semaphore_signal(second_barrier, inc=1, device_id=(nbr,),
                                    device_id_type=pl.DeviceIdType.MESH)
            pl.semaphore_wait(second_barrier, 2)


def kernel(x, Wq, K_ext, V_ext, Wo):
    my = lax.axis_index("i")
    x2 = x.reshape(SQ, DMODEL).astype(jnp.bfloat16)
    wq_s = lax.dynamic_slice(
        Wq, (0, my * HQ_LOCAL * DH), (DMODEL, HQ_LOCAL * DH)
    ).astype(jnp.bfloat16)
    wo_s = lax.dynamic_slice(
        Wo, (my * HQ_LOCAL * DH, 0), (HQ_LOCAL * DH, DMODEL)
    ).astype(jnp.bfloat16)
    k = jnp.transpose(K_ext.reshape(SKV, HQ_LOCAL, DH), (1, 0, 2)).astype(
        jnp.bfloat16)
    v = jnp.transpose(V_ext.reshape(SKV, HQ_LOCAL, DH), (1, 0, 2)).astype(
        jnp.bfloat16)

    out = pl.pallas_call(
        _body,
        grid=(N_QB,),
        in_specs=[
            pl.BlockSpec((SQ, DMODEL), lambda j: (0, 0)),
            pl.BlockSpec((DMODEL, HQ_LOCAL * DH), lambda j: (0, 0)),
            pl.BlockSpec((HQ_LOCAL, SKV, DH), lambda j: (0, 0, 0)),
            pl.BlockSpec((HQ_LOCAL, SKV, DH), lambda j: (0, 0, 0)),
            pl.BlockSpec((HQ_LOCAL * DH, DMODEL), lambda j: (0, 0)),
        ],
        out_specs=pl.BlockSpec((SQ, DMODEL), lambda j: (0, 0)),
        out_shape=jax.ShapeDtypeStruct((SQ, DMODEL), jnp.float32),
        scratch_shapes=[
            pltpu.VMEM((SQ, DMODEL), jnp.float32),
            pltpu.VMEM((QBLK, HQ_LOCAL * DH), jnp.bfloat16),
            pltpu.VMEM((N_DEV - 1, CHUNK, DMODEL), jnp.bfloat16),
            pltpu.VMEM((N_DEV - 1, CHUNK, DMODEL), jnp.bfloat16),
            pltpu.VMEM((N_DEV, CHUNK, DMODEL), jnp.bfloat16),
            pltpu.SemaphoreType.DMA((N_DEV - 1,)),
            pltpu.SemaphoreType.DMA((N_DEV - 1,)),
            pltpu.SemaphoreType.DMA((N_DEV - 1,)),
            pltpu.SemaphoreType.DMA((N_DEV - 1,)),
        ],
        compiler_params=pltpu.CompilerParams(
            dimension_semantics=("arbitrary",),
            **({"collective_id": 0} if _COMM else {}),
        ),
    )(x2, wq_s, k, v, wo_s)
    return out.reshape(1, SQ, DMODEL)


# device time: 99506 ns/iter; 1.6483x vs baseline; 1.0546x over previous
import functools
import os

import jax
import jax.numpy as jnp
from jax import lax
from jax.experimental import pallas as pl
from jax.experimental.pallas import tpu as pltpu

_COMM = os.environ.get("KERNEL_NO_COMM", "0") != "1"

N_DEV = 4
SQ = 2048
SKV = 2048
HQ_LOCAL = 8
DH = 128
DMODEL = 1024
QBLK = 512
N_QB = SQ // QBLK
CHUNK = SQ // N_DEV
SCALE = 0.08838834764831843
LOCAL_WINDOW = 128
GLOBAL_TOKENS = 32
GBLK = 32
WWIN = 768


def _chunk_rows(c):
    return pl.ds(c * CHUNK, CHUNK)


def _body(x_ref, wq_ref, k_ref, v_ref, wo_ref, out_ref,
          partial_ref, ctx_ref, rs_send_buf, rs_recv_ref, ag_buf,
          rs_send_sems, rs_recv_sems, ag_send_sems, ag_recv_sems):
    j = pl.program_id(0)
    my = lax.axis_index("i")
    left = lax.rem(my + N_DEV - 1, N_DEV)
    right = lax.rem(my + 1, N_DEV)
    qb = lax.rem(my - j + 2 * N_DEV, N_DEV)
    rows = _chunk_rows(qb)

    q_all = (lax.dot_general(
        x_ref[rows, :], wq_ref[...],
        (((1,), (0,)), ((), ())), preferred_element_type=jnp.float32
    ) * SCALE).astype(jnp.bfloat16)

    w0 = jnp.clip(qb * QBLK - LOCAL_WINDOW, 0, SKV - WWIN)
    w0 = pl.multiple_of(w0, 128)
    qi_w = qb * QBLK + lax.broadcasted_iota(jnp.int32, (QBLK, WWIN), 0)
    ki_w = w0 + lax.broadcasted_iota(jnp.int32, (QBLK, WWIN), 1)
    bias_w = jnp.where(
        (jnp.abs(qi_w - ki_w) <= LOCAL_WINDOW) | (ki_w < GLOBAL_TOKENS),
        0.0, -1e9).astype(jnp.float32)
    bias_g = jnp.where(qb == 0, -1e9, 0.0).astype(jnp.float32)

    for h in range(HQ_LOCAL):
        hcols = pl.ds(h * DH, DH)
        qh = q_all[:, h * DH:(h + 1) * DH]
        s_g = lax.dot_general(
            qh, k_ref[h, 0:GBLK, :], (((1,), (1,)), ((), ())),
            preferred_element_type=jnp.float32) + bias_g
        s_w = lax.dot_general(
            qh, k_ref[h, pl.ds(w0, WWIN), :], (((1,), (1,)), ((), ())),
            preferred_element_type=jnp.float32) + bias_w

        e_g = jnp.exp(s_g)
        e_w = jnp.exp(s_w)
        inv = 1.0 / (e_g.sum(axis=1, keepdims=True)
                     + e_w.sum(axis=1, keepdims=True))
        ctx = lax.dot_general(
            (e_g * inv).astype(jnp.bfloat16), v_ref[h, 0:GBLK, :],
            (((1,), (0,)), ((), ())), preferred_element_type=jnp.float32)
        ctx = ctx + lax.dot_general(
            (e_w * inv).astype(jnp.bfloat16), v_ref[h, pl.ds(w0, WWIN), :],
            (((1,), (0,)), ((), ())), preferred_element_type=jnp.float32)
        ctx_ref[:, hcols] = ctx.astype(jnp.bfloat16)

    @pl.when(qb == 0)
    def _():
        for h in range(HQ_LOCAL):
            hcols = pl.ds(h * DH, DH)
            q32 = q_all[0:32, h * DH:(h + 1) * DH]
            s32 = lax.dot_general(
                q32, k_ref[h], (((1,), (1,)), ((), ())),
                preferred_element_type=jnp.float32)
            e32 = jnp.exp(s32)
            w32 = (e32 / e32.sum(axis=1, keepdims=True)).astype(jnp.bfloat16)
            ctx32 = lax.dot_general(
                w32, v_ref[h], (((1,), (0,)), ((), ())),
                preferred_element_type=jnp.float32)
            ctx_ref[0:32, hcols] = ctx32.astype(jnp.bfloat16)

    partial_ref[rows, :] = lax.dot_general(
        ctx_ref[...], wo_ref[...], (((1,), (0,)), ((), ())),
        preferred_element_type=jnp.float32)

    def rs_copy(step):
        return pltpu.make_async_remote_copy(
            src_ref=rs_send_buf.at[step],
            dst_ref=rs_recv_ref.at[step],
            send_sem=rs_send_sems.at[step],
            recv_sem=rs_recv_sems.at[step],
            device_id=(right,),
            device_id_type=pl.DeviceIdType.MESH,
        )

    def rs_send(step):
        c = lax.rem(my - step + 2 * N_DEV, N_DEV)
        rs_send_buf[step] = partial_ref[_chunk_rows(c), :].astype(jnp.bfloat16)
        rs_copy(step).start()

    def rs_wait_and_add(step):
        c = lax.rem(my - step - 1 + 2 * N_DEV, N_DEV)
        rs_copy(step).wait_recv()
        rrows = _chunk_rows(c)
        partial_ref[rrows, :] = partial_ref[rrows, :] + rs_recv_ref[step]
        rs_copy(step).wait_send()

    def ag_copy(c, sem_idx, target, half=None):
        if half is None:
            src = ag_buf.at[c]
        elif half == 0:
            src = ag_buf.at[c, 0:CHUNK // 2, :]
        else:
            src = ag_buf.at[c, CHUNK // 2:CHUNK, :]
        return pltpu.make_async_remote_copy(
            src_ref=src,
            dst_ref=src,
            send_sem=ag_send_sems.at[sem_idx],
            recv_sem=ag_recv_sems.at[sem_idx],
            device_id=(target,),
            device_id_type=pl.DeviceIdType.MESH,
        )

    if not _COMM:
        @pl.when(j == N_QB - 1)
        def _():
            out_ref[...] = partial_ref[...]
        return

    @pl.when(j == 0)
    def _():
        barrier_sem = pltpu.get_barrier_semaphore()
        for nbr in (left, right):
            pl.semaphore_signal(barrier_sem, inc=1, device_id=(nbr,),
                                device_id_type=pl.DeviceIdType.MESH)
        pl.semaphore_wait(barrier_sem, 2)
        rs_send(0)

    for step in range(N_QB - 2):
        @pl.when(j == step + 1)
        def _():
            rs_wait_and_add(step)
            rs_send(step + 1)

    @pl.when(j == N_QB - 1)
    def _():
        rs_wait_and_add(N_DEV - 2)

        c_own = lax.rem(my + 1, N_DEV)
        orows = _chunk_rows(c_own)
        ag_buf[c_own] = partial_ref[orows, :].astype(jnp.bfloat16)

        send_r = ag_copy(c_own, 0, right)
        send_l = ag_copy(c_own, 1, left)
        send_r.start()
        send_l.start()

        out_ref[orows, :] = partial_ref[orows, :]

        ag_copy(my, 0, right).wait_recv()
        fwd_a = ag_copy(my, 2, right, half=0)
        fwd_a.start()
        out_ref[_chunk_rows(my), :] = ag_buf[my].astype(jnp.float32)

        c_r = lax.rem(my + 2, N_DEV)
        ag_copy(c_r, 1, right).wait_recv()
        fwd_b = ag_copy(c_r, 3, left, half=1)
        fwd_b.start()
        out_ref[_chunk_rows(c_r), :] = ag_buf[c_r].astype(jnp.float32)

        ag_copy(left, 2, right, half=0).wait_recv()
        ag_copy(left, 3, left, half=1).wait_recv()
        out_ref[_chunk_rows(left), :] = ag_buf[left].astype(jnp.float32)

        send_r.wait_send()
        send_l.wait_send()
        fwd_a.wait_send()
        fwd_b.wait_send()

        @functools.partial(pl.run_scoped,
                           second_barrier=pltpu.SemaphoreType.REGULAR)
        def _(second_barrier):
            for nbr in (left, right):
                pl.semaphore_signal(second_barrier, inc=1, device_id=(nbr,),
                                    device_id_type=pl.DeviceIdType.MESH)
            pl.semaphore_wait(second_barrier, 2)


def kernel(x, Wq, K_ext, V_ext, Wo):
    my = lax.axis_index("i")
    x2 = x.reshape(SQ, DMODEL).astype(jnp.bfloat16)
    wq_s = lax.dynamic_slice(
        Wq, (0, my * HQ_LOCAL * DH), (DMODEL, HQ_LOCAL * DH)
    ).astype(jnp.bfloat16)
    wo_s = lax.dynamic_slice(
        Wo, (my * HQ_LOCAL * DH, 0), (HQ_LOCAL * DH, DMODEL)
    ).astype(jnp.bfloat16)
    k = jnp.transpose(K_ext.reshape(SKV, HQ_LOCAL, DH), (1, 0, 2)).astype(
        jnp.bfloat16)
    v = jnp.transpose(V_ext.reshape(SKV, HQ_LOCAL, DH), (1, 0, 2)).astype(
        jnp.bfloat16)

    out = pl.pallas_call(
        _body,
        grid=(N_QB,),
        in_specs=[
            pl.BlockSpec((SQ, DMODEL), lambda j: (0, 0)),
            pl.BlockSpec((DMODEL, HQ_LOCAL * DH), lambda j: (0, 0)),
            pl.BlockSpec((HQ_LOCAL, SKV, DH), lambda j: (0, 0, 0)),
            pl.BlockSpec((HQ_LOCAL, SKV, DH), lambda j: (0, 0, 0)),
            pl.BlockSpec((HQ_LOCAL * DH, DMODEL), lambda j: (0, 0)),
        ],
        out_specs=pl.BlockSpec((SQ, DMODEL), lambda j: (0, 0)),
        out_shape=jax.ShapeDtypeStruct((SQ, DMODEL), jnp.float32),
        scratch_shapes=[
            pltpu.VMEM((SQ, DMODEL), jnp.float32),
            pltpu.VMEM((QBLK, HQ_LOCAL * DH), jnp.bfloat16),
            pltpu.VMEM((N_DEV - 1, CHUNK, DMODEL), jnp.bfloat16),
            pltpu.VMEM((N_DEV - 1, CHUNK, DMODEL), jnp.bfloat16),
            pltpu.VMEM((N_DEV, CHUNK, DMODEL), jnp.bfloat16),
            pltpu.SemaphoreType.DMA((N_DEV - 1,)),
            pltpu.SemaphoreType.DMA((N_DEV - 1,)),
            pltpu.SemaphoreType.DMA((N_DEV,)),
            pltpu.SemaphoreType.DMA((N_DEV,)),
        ],
        compiler_params=pltpu.CompilerParams(
            dimension_semantics=("arbitrary",),
            **({"collective_id": 0} if _COMM else {}),
        ),
    )(x2, wq_s, k, v, wo_s)
    return out.reshape(1, SQ, DMODEL)


# device time: 99408 ns/iter; 1.6500x vs baseline; 1.0010x over previous
import functools
import os

import jax
import jax.numpy as jnp
from jax import lax
from jax.experimental import pallas as pl
from jax.experimental.pallas import tpu as pltpu

_COMM = os.environ.get("KERNEL_NO_COMM", "0") != "1"

N_DEV = 4
SQ = 2048
SKV = 2048
HQ_LOCAL = 8
DH = 128
DMODEL = 1024
QBLK = 512
N_QB = SQ // QBLK
CHUNK = SQ // N_DEV
SCALE = 0.08838834764831843
LOCAL_WINDOW = 128
GLOBAL_TOKENS = 32
GBLK = 32
WWIN = 768


def _chunk_rows(c):
    return pl.ds(c * CHUNK, CHUNK)


def _body(x_ref, wq_ref, k_ref, v_ref, wo_ref, out_ref,
          partial_ref, ctx_ref, rs_send_buf, rs_recv_ref, ag_buf,
          rs_send_sems, rs_recv_sems, ag_send_sems, ag_recv_sems):
    j = pl.program_id(0)
    my = lax.axis_index("i")
    left = lax.rem(my + N_DEV - 1, N_DEV)
    right = lax.rem(my + 1, N_DEV)
    qb = lax.rem(my - j + 2 * N_DEV, N_DEV)
    rows = _chunk_rows(qb)

    q_all = (lax.dot_general(
        x_ref[rows, :], wq_ref[...],
        (((1,), (0,)), ((), ())), preferred_element_type=jnp.float32
    ) * SCALE).astype(jnp.bfloat16)

    w0 = jnp.clip(qb * QBLK - LOCAL_WINDOW, 0, SKV - WWIN)
    w0 = pl.multiple_of(w0, 128)
    qi_w = qb * QBLK + lax.broadcasted_iota(jnp.int32, (QBLK, WWIN), 0)
    ki_w = w0 + lax.broadcasted_iota(jnp.int32, (QBLK, WWIN), 1)
    bias_w = jnp.where(
        (jnp.abs(qi_w - ki_w) <= LOCAL_WINDOW) | (ki_w < GLOBAL_TOKENS),
        0.0, -1e9).astype(jnp.float32)
    bias_g = jnp.where(qb == 0, -1e9, 0.0).astype(jnp.float32)

    for h in range(HQ_LOCAL):
        hcols = slice(h * DH, (h + 1) * DH)
        qh = q_all[:, hcols]
        s_g = lax.dot_general(
            qh, k_ref[0:GBLK, hcols], (((1,), (1,)), ((), ())),
            preferred_element_type=jnp.float32) + bias_g
        s_w = lax.dot_general(
            qh, k_ref[pl.ds(w0, WWIN), hcols], (((1,), (1,)), ((), ())),
            preferred_element_type=jnp.float32) + bias_w

        e_g = jnp.exp(s_g)
        e_w = jnp.exp(s_w)
        inv = 1.0 / (e_g.sum(axis=1, keepdims=True)
                     + e_w.sum(axis=1, keepdims=True))
        ctx = lax.dot_general(
            (e_g * inv).astype(jnp.bfloat16), v_ref[0:GBLK, hcols],
            (((1,), (0,)), ((), ())), preferred_element_type=jnp.float32)
        ctx = ctx + lax.dot_general(
            (e_w * inv).astype(jnp.bfloat16), v_ref[pl.ds(w0, WWIN), hcols],
            (((1,), (0,)), ((), ())), preferred_element_type=jnp.float32)
        ctx_ref[:, hcols] = ctx.astype(jnp.bfloat16)

    @pl.when(qb == 0)
    def _():
        for h in range(HQ_LOCAL):
            hcols = slice(h * DH, (h + 1) * DH)
            q32 = q_all[0:32, hcols]
            s32 = lax.dot_general(
                q32, k_ref[:, hcols], (((1,), (1,)), ((), ())),
                preferred_element_type=jnp.float32)
            e32 = jnp.exp(s32)
            w32 = (e32 / e32.sum(axis=1, keepdims=True)).astype(jnp.bfloat16)
            ctx32 = lax.dot_general(
                w32, v_ref[:, hcols], (((1,), (0,)), ((), ())),
                preferred_element_type=jnp.float32)
            ctx_ref[0:32, hcols] = ctx32.astype(jnp.bfloat16)

    partial_ref[rows, :] = lax.dot_general(
        ctx_ref[...], wo_ref[...], (((1,), (0,)), ((), ())),
        preferred_element_type=jnp.float32)

    def rs_copy(step):
        return pltpu.make_async_remote_copy(
            src_ref=rs_send_buf.at[step],
            dst_ref=rs_recv_ref.at[step],
            send_sem=rs_send_sems.at[step],
            recv_sem=rs_recv_sems.at[step],
            device_id=(right,),
            device_id_type=pl.DeviceIdType.MESH,
        )

    def rs_send(step):
        c = lax.rem(my - step + 2 * N_DEV, N_DEV)
        rs_send_buf[step] = partial_ref[_chunk_rows(c), :].astype(jnp.bfloat16)
        rs_copy(step).start()

    def rs_wait_and_add(step):
        c = lax.rem(my - step - 1 + 2 * N_DEV, N_DEV)
        rs_copy(step).wait_recv()
        rrows = _chunk_rows(c)
        partial_ref[rrows, :] = partial_ref[rrows, :] + rs_recv_ref[step]
        rs_copy(step).wait_send()

    def ag_copy(c, sem_idx, target, half=None):
        if half is None:
            src = ag_buf.at[c]
        elif half == 0:
            src = ag_buf.at[c, 0:CHUNK // 2, :]
        else:
            src = ag_buf.at[c, CHUNK // 2:CHUNK, :]
        return pltpu.make_async_remote_copy(
            src_ref=src,
            dst_ref=src,
            send_sem=ag_send_sems.at[sem_idx],
            recv_sem=ag_recv_sems.at[sem_idx],
            device_id=(target,),
            device_id_type=pl.DeviceIdType.MESH,
        )

    if not _COMM:
        @pl.when(j == N_QB - 1)
        def _():
            out_ref[...] = partial_ref[...]
        return

    @pl.when(j == 0)
    def _():
        barrier_sem = pltpu.get_barrier_semaphore()
        for nbr in (left, right):
            pl.semaphore_signal(barrier_sem, inc=1, device_id=(nbr,),
                                device_id_type=pl.DeviceIdType.MESH)
        pl.semaphore_wait(barrier_sem, 2)
        rs_send(0)

    for step in range(N_QB - 2):
        @pl.when(j == step + 1)
        def _():
            rs_wait_and_add(step)
            rs_send(step + 1)

    @pl.when(j == N_QB - 1)
    def _():
        rs_wait_and_add(N_DEV - 2)

        c_own = lax.rem(my + 1, N_DEV)
        orows = _chunk_rows(c_own)
        ag_buf[c_own] = partial_ref[orows, :].astype(jnp.bfloat16)

        send_r = ag_copy(c_own, 0, right)
        send_l = ag_copy(c_own, 1, left)
        send_r.start()
        send_l.start()

        out_ref[orows, :] = partial_ref[orows, :]

        ag_copy(my, 0, right).wait_recv()
        fwd_a = ag_copy(my, 2, right, half=0)
        fwd_a.start()
        out_ref[_chunk_rows(my), :] = ag_buf[my].astype(jnp.float32)

        c_r = lax.rem(my + 2, N_DEV)
        ag_copy(c_r, 1, right).wait_recv()
        fwd_b = ag_copy(c_r, 3, left, half=1)
        fwd_b.start()
        out_ref[_chunk_rows(c_r), :] = ag_buf[c_r].astype(jnp.float32)

        ag_copy(left, 2, right, half=0).wait_recv()
        ag_copy(left, 3, left, half=1).wait_recv()
        out_ref[_chunk_rows(left), :] = ag_buf[left].astype(jnp.float32)

        send_r.wait_send()
        send_l.wait_send()
        fwd_a.wait_send()
        fwd_b.wait_send()

        @functools.partial(pl.run_scoped,
                           second_barrier=pltpu.SemaphoreType.REGULAR)
        def _(second_barrier):
            for nbr in (left, right):
                pl.semaphore_signal(second_barrier, inc=1, device_id=(nbr,),
                                    device_id_type=pl.DeviceIdType.MESH)
            pl.semaphore_wait(second_barrier, 2)


def kernel(x, Wq, K_ext, V_ext, Wo):
    my = lax.axis_index("i")
    x2 = x.reshape(SQ, DMODEL).astype(jnp.bfloat16)
    wq_s = lax.dynamic_slice(
        Wq, (0, my * HQ_LOCAL * DH), (DMODEL, HQ_LOCAL * DH)
    ).astype(jnp.bfloat16)
    wo_s = lax.dynamic_slice(
        Wo, (my * HQ_LOCAL * DH, 0), (HQ_LOCAL * DH, DMODEL)
    ).astype(jnp.bfloat16)
    k = K_ext.reshape(SKV, HQ_LOCAL * DH).astype(jnp.bfloat16)
    v = V_ext.reshape(SKV, HQ_LOCAL * DH).astype(jnp.bfloat16)

    out = pl.pallas_call(
        _body,
        grid=(N_QB,),
        in_specs=[
            pl.BlockSpec((SQ, DMODEL), lambda j: (0, 0)),
            pl.BlockSpec((DMODEL, HQ_LOCAL * DH), lambda j: (0, 0)),
            pl.BlockSpec((SKV, HQ_LOCAL * DH), lambda j: (0, 0)),
            pl.BlockSpec((SKV, HQ_LOCAL * DH), lambda j: (0, 0)),
            pl.BlockSpec((HQ_LOCAL * DH, DMODEL), lambda j: (0, 0)),
        ],
        out_specs=pl.BlockSpec((SQ, DMODEL), lambda j: (0, 0)),
        out_shape=jax.ShapeDtypeStruct((SQ, DMODEL), jnp.float32),
        scratch_shapes=[
            pltpu.VMEM((SQ, DMODEL), jnp.float32),
            pltpu.VMEM((QBLK, HQ_LOCAL * DH), jnp.bfloat16),
            pltpu.VMEM((N_DEV - 1, CHUNK, DMODEL), jnp.bfloat16),
            pltpu.VMEM((N_DEV - 1, CHUNK, DMODEL), jnp.bfloat16),
            pltpu.VMEM((N_DEV, CHUNK, DMODEL), jnp.bfloat16),
            pltpu.SemaphoreType.DMA((N_DEV - 1,)),
            pltpu.SemaphoreType.DMA((N_DEV - 1,)),
            pltpu.SemaphoreType.DMA((N_DEV,)),
            pltpu.SemaphoreType.DMA((N_DEV,)),
        ],
        compiler_params=pltpu.CompilerParams(
            dimension_semantics=("arbitrary",),
            **({"collective_id": 0} if _COMM else {}),
        ),
    )(x2, wq_s, k, v, wo_s)
    return out.reshape(1, SQ, DMODEL)


# device time: 98825 ns/iter; 1.6597x vs baseline; 1.0059x over previous
import functools
import os

import jax
import jax.numpy as jnp
from jax import lax
from jax.experimental import pallas as pl
from jax.experimental.pallas import tpu as pltpu

_COMM = os.environ.get("KERNEL_NO_COMM", "0") != "1"
_AG = os.environ.get("KERNEL_NO_AG", "0") != "1"
_exp = (lambda s: s + 1.0) if os.environ.get("KERNEL_NO_EXP") == "1" else __import__('jax.numpy', fromlist=['exp']).exp

N_DEV = 4
SQ = 2048
SKV = 2048
HQ_LOCAL = 8
DH = 128
DMODEL = 1024
QBLK = 512
N_QB = SQ // QBLK
CHUNK = SQ // N_DEV
SCALE = 0.08838834764831843
LOCAL_WINDOW = 128
GLOBAL_TOKENS = 32
GBLK = 32
WWIN = 768


def _chunk_rows(c):
    return pl.ds(c * CHUNK, CHUNK)


def _body(x_ref, wq_ref, k_ref, v_ref, wo_ref, out_ref,
          partial_ref, ctx_ref, rs_recv_ref,
          rs_send_sems, rs_recv_sems, ag_send_sems, ag_recv_sems):
    j = pl.program_id(0)
    my = lax.axis_index("i")
    left = lax.rem(my + N_DEV - 1, N_DEV)
    right = lax.rem(my + 1, N_DEV)
    qb = lax.rem(my - j + 2 * N_DEV, N_DEV)
    rows = _chunk_rows(qb)

    q_all = (lax.dot_general(
        x_ref[rows, :], wq_ref[...],
        (((1,), (0,)), ((), ())), preferred_element_type=jnp.float32
    ) * SCALE).astype(jnp.bfloat16)

    w0 = jnp.clip(qb * QBLK - LOCAL_WINDOW, 0, SKV - WWIN)
    w0 = pl.multiple_of(w0, 128)
    qi_w = qb * QBLK + lax.broadcasted_iota(jnp.int32, (QBLK, WWIN), 0)
    ki_w = w0 + lax.broadcasted_iota(jnp.int32, (QBLK, WWIN), 1)
    bias_w = jnp.where(
        (jnp.abs(qi_w - ki_w) <= LOCAL_WINDOW) | (ki_w < GLOBAL_TOKENS),
        0.0, -1e9).astype(jnp.float32)
    bias_g = jnp.where(qb == 0, -1e9, 0.0).astype(jnp.float32)

    for h in range(HQ_LOCAL):
        hcols = slice(h * DH, (h + 1) * DH)
        qh = q_all[:, hcols]
        s_g = lax.dot_general(
            qh, k_ref[0:GBLK, hcols], (((1,), (1,)), ((), ())),
            preferred_element_type=jnp.float32) + bias_g
        s_w = lax.dot_general(
            qh, k_ref[pl.ds(w0, WWIN), hcols], (((1,), (1,)), ((), ())),
            preferred_element_type=jnp.float32) + bias_w

        e_g = _exp(s_g)
        e_w = _exp(s_w)
        inv = 1.0 / (e_g.sum(axis=1, keepdims=True)
                     + e_w.sum(axis=1, keepdims=True))
        ctx = lax.dot_general(
            (e_g * inv).astype(jnp.bfloat16), v_ref[0:GBLK, hcols],
            (((1,), (0,)), ((), ())), preferred_element_type=jnp.float32)
        ctx = ctx + lax.dot_general(
            (e_w * inv).astype(jnp.bfloat16), v_ref[pl.ds(w0, WWIN), hcols],
            (((1,), (0,)), ((), ())), preferred_element_type=jnp.float32)
        ctx_ref[:, hcols] = ctx.astype(jnp.bfloat16)

    @pl.when(qb == 0)
    def _():
        for h in range(HQ_LOCAL):
            hcols = slice(h * DH, (h + 1) * DH)
            q32 = q_all[0:32, hcols]
            s32 = lax.dot_general(
                q32, k_ref[:, hcols], (((1,), (1,)), ((), ())),
                preferred_element_type=jnp.float32)
            e32 = _exp(s32)
            w32 = (e32 / e32.sum(axis=1, keepdims=True)).astype(jnp.bfloat16)
            ctx32 = lax.dot_general(
                w32, v_ref[:, hcols], (((1,), (0,)), ((), ())),
                preferred_element_type=jnp.float32)
            ctx_ref[0:32, hcols] = ctx32.astype(jnp.bfloat16)

    proj = lax.dot_general(
        ctx_ref[...], wo_ref[...], (((1,), (0,)), ((), ())),
        preferred_element_type=jnp.float32)

    def rs_copy(step, chunk_idx):
        return pltpu.make_async_remote_copy(
            src_ref=partial_ref.at[_chunk_rows(chunk_idx)],
            dst_ref=rs_recv_ref.at[step],
            send_sem=rs_send_sems.at[step],
            recv_sem=rs_recv_sems.at[step],
            device_id=(right,),
            device_id_type=pl.DeviceIdType.MESH,
        )

    def ag_copy(c, sem_idx, target, half=None):
        if half is None:
            src = partial_ref.at[_chunk_rows(c)]
        elif half == 0:
            src = partial_ref.at[pl.ds(c * CHUNK, CHUNK // 2)]
        else:
            src = partial_ref.at[pl.ds(c * CHUNK + CHUNK // 2, CHUNK // 2)]
        return pltpu.make_async_remote_copy(
            src_ref=src,
            dst_ref=src,
            send_sem=ag_send_sems.at[sem_idx],
            recv_sem=ag_recv_sems.at[sem_idx],
            device_id=(target,),
            device_id_type=pl.DeviceIdType.MESH,
        )

    if not _COMM:
        partial_ref[rows, :] = proj.astype(jnp.bfloat16)

        @pl.when(j == N_QB - 1)
        def _():
            out_ref[...] = partial_ref[...].astype(jnp.float32)
        return

    @pl.when(j == 0)
    def _():
        partial_ref[rows, :] = proj.astype(jnp.bfloat16)
        barrier_sem = pltpu.get_barrier_semaphore()
        for nbr in (left, right):
            pl.semaphore_signal(barrier_sem, inc=1, device_id=(nbr,),
                                device_id_type=pl.DeviceIdType.MESH)
        pl.semaphore_wait(barrier_sem, 2)

    @pl.when(j != 0)
    def _():
        step = j - 1
        rs_copy(step, qb).wait_recv()
        partial_ref[rows, :] = (proj + rs_recv_ref[step]).astype(jnp.bfloat16)
        rs_copy(step, qb).wait_send()

    @pl.when(j != N_QB - 1)
    def _():
        rs_copy(j, lax.rem(my - j + 2 * N_DEV, N_DEV)).start()

    @pl.when(j == N_QB - 1)
    def _():
        if not _AG:
            out_ref[...] = partial_ref[...].astype(jnp.float32)
            return

        c_own = lax.rem(my + 1, N_DEV)
        orows = _chunk_rows(c_own)

        send_r = ag_copy(c_own, 0, right)
        send_l = ag_copy(c_own, 1, left)
        send_r.start()
        send_l.start()

        out_ref[orows, :] = partial_ref[orows, :].astype(jnp.float32)

        ag_copy(my, 0, right).wait_recv()
        fwd_a = ag_copy(my, 2, right, half=0)
        fwd_a.start()
        out_ref[_chunk_rows(my), :] = partial_ref[_chunk_rows(my), :].astype(
            jnp.float32)

        c_r = lax.rem(my + 2, N_DEV)
        ag_copy(c_r, 1, right).wait_recv()
        fwd_b = ag_copy(c_r, 3, left, half=1)
        fwd_b.start()
        out_ref[_chunk_rows(c_r), :] = partial_ref[_chunk_rows(c_r), :].astype(
            jnp.float32)

        ag_copy(left, 2, right, half=0).wait_recv()
        ag_copy(left, 3, left, half=1).wait_recv()
        out_ref[_chunk_rows(left), :] = partial_ref[_chunk_rows(left), :].astype(
            jnp.float32)

        send_r.wait_send()
        send_l.wait_send()
        fwd_a.wait_send()
        fwd_b.wait_send()

        @functools.partial(pl.run_scoped,
                           second_barrier=pltpu.SemaphoreType.REGULAR)
        def _(second_barrier):
            for nbr in (left, right):
                pl.semaphore_signal(second_barrier, inc=1, device_id=(nbr,),
                                    device_id_type=pl.DeviceIdType.MESH)
            pl.semaphore_wait(second_barrier, 2)


def kernel(x, Wq, K_ext, V_ext, Wo):
    my = lax.axis_index("i")
    x2 = x.reshape(SQ, DMODEL).astype(jnp.bfloat16)
    wq_s = lax.dynamic_slice(
        Wq, (0, my * HQ_LOCAL * DH), (DMODEL, HQ_LOCAL * DH)
    ).astype(jnp.bfloat16)
    wo_s = lax.dynamic_slice(
        Wo, (my * HQ_LOCAL * DH, 0), (HQ_LOCAL * DH, DMODEL)
    ).astype(jnp.bfloat16)
    k = K_ext.reshape(SKV, HQ_LOCAL * DH).astype(jnp.bfloat16)
    v = V_ext.reshape(SKV, HQ_LOCAL * DH).astype(jnp.bfloat16)

    out = pl.pallas_call(
        _body,
        grid=(N_QB,),
        in_specs=[
            pl.BlockSpec((SQ, DMODEL), lambda j: (0, 0)),
            pl.BlockSpec((DMODEL, HQ_LOCAL * DH), lambda j: (0, 0)),
            pl.BlockSpec((SKV, HQ_LOCAL * DH), lambda j: (0, 0)),
            pl.BlockSpec((SKV, HQ_LOCAL * DH), lambda j: (0, 0)),
            pl.BlockSpec((HQ_LOCAL * DH, DMODEL), lambda j: (0, 0)),
        ],
        out_specs=pl.BlockSpec((SQ, DMODEL), lambda j: (0, 0)),
        out_shape=jax.ShapeDtypeStruct((SQ, DMODEL), jnp.float32),
        scratch_shapes=[
            pltpu.VMEM((SQ, DMODEL), jnp.bfloat16),
            pltpu.VMEM((QBLK, HQ_LOCAL * DH), jnp.bfloat16),
            pltpu.VMEM((N_DEV - 1, CHUNK, DMODEL), jnp.bfloat16),
            pltpu.SemaphoreType.DMA((N_DEV - 1,)),
            pltpu.SemaphoreType.DMA((N_DEV - 1,)),
            pltpu.SemaphoreType.DMA((N_DEV,)),
            pltpu.SemaphoreType.DMA((N_DEV,)),
        ],
        compiler_params=pltpu.CompilerParams(
            dimension_semantics=("arbitrary",),
            **({"collective_id": 0} if _COMM else {}),
        ),
    )(x2, wq_s, k, v, wo_s)
    return out.reshape(1, SQ, DMODEL)


# device time: 93705 ns/iter; 1.7504x vs baseline; 1.0546x over previous
import functools
import os

import jax
import jax.numpy as jnp
from jax import lax
from jax.experimental import pallas as pl
from jax.experimental.pallas import tpu as pltpu

_COMM = os.environ.get("KERNEL_NO_COMM", "0") != "1"
_exp = (lambda s: s + 1.0) if os.environ.get("KERNEL_NO_EXP") == "1" else jnp.exp

N_DEV = 4
SQ = 2048
SKV = 2048
HQ_LOCAL = 8
DH = 128
DMODEL = 1024
N_QB = 4
CHUNK = SQ // N_DEV
HBLK = CHUNK // 2
N_T = 2 * N_QB
SCALE = 0.08838834764831843
LOCAL_WINDOW = 128
GLOBAL_TOKENS = 32
GBLK = 32
WWIN = 640


def _chunk_rows(c):
    return pl.ds(c * CHUNK, CHUNK)


def _body(x_ref, wq_ref, k_ref, v_ref, wo_ref, out_ref,
          partial_ref, ctx_ref, rs_recv_ref,
          rs_send_sems, rs_recv_sems, ag_send_sems, ag_recv_sems):
    t = pl.program_id(0)
    my = lax.axis_index("i")
    left = lax.rem(my + N_DEV - 1, N_DEV)
    right = lax.rem(my + 1, N_DEV)
    hop = lax.div(t, 2)
    half = lax.rem(t, 2)
    qb = lax.rem(my - hop + 2 * N_DEV, N_DEV)
    r0 = qb * CHUNK + half * HBLK
    rows = pl.ds(r0, HBLK)

    q_all = (lax.dot_general(
        x_ref[rows, :], wq_ref[...],
        (((1,), (0,)), ((), ())), preferred_element_type=jnp.float32
    ) * SCALE).astype(jnp.bfloat16)

    w0 = jnp.clip(r0 - LOCAL_WINDOW, 0, SKV - WWIN)
    w0 = pl.multiple_of(w0, 128)
    qi_w = r0 + lax.broadcasted_iota(jnp.int32, (HBLK, WWIN), 0)
    ki_w = w0 + lax.broadcasted_iota(jnp.int32, (HBLK, WWIN), 1)
    bias_w = jnp.where(
        (jnp.abs(qi_w - ki_w) <= LOCAL_WINDOW) | (ki_w < GLOBAL_TOKENS),
        0.0, -1e9).astype(jnp.float32)
    bias_g = jnp.where(w0 == 0, -1e9, 0.0).astype(jnp.float32)

    for h in range(HQ_LOCAL):
        hcols = slice(h * DH, (h + 1) * DH)
        qh = q_all[:, hcols]
        s_g = lax.dot_general(
            qh, k_ref[0:GBLK, hcols], (((1,), (1,)), ((), ())),
            preferred_element_type=jnp.float32) + bias_g
        s_w = lax.dot_general(
            qh, k_ref[pl.ds(w0, WWIN), hcols], (((1,), (1,)), ((), ())),
            preferred_element_type=jnp.float32) + bias_w

        e_g = _exp(s_g)
        e_w = _exp(s_w)
        inv = 1.0 / (e_g.sum(axis=1, keepdims=True)
                     + e_w.sum(axis=1, keepdims=True))
        ctx = lax.dot_general(
            (e_g * inv).astype(jnp.bfloat16), v_ref[0:GBLK, hcols],
            (((1,), (0,)), ((), ())), preferred_element_type=jnp.float32)
        ctx = ctx + lax.dot_general(
            (e_w * inv).astype(jnp.bfloat16), v_ref[pl.ds(w0, WWIN), hcols],
            (((1,), (0,)), ((), ())), preferred_element_type=jnp.float32)
        ctx_ref[:, hcols] = ctx.astype(jnp.bfloat16)

    @pl.when(r0 == 0)
    def _():
        for h in range(HQ_LOCAL):
            hcols = slice(h * DH, (h + 1) * DH)
            q32 = q_all[0:GLOBAL_TOKENS, hcols]
            s32 = lax.dot_general(
                q32, k_ref[:, hcols], (((1,), (1,)), ((), ())),
                preferred_element_type=jnp.float32)
            e32 = _exp(s32)
            w32 = (e32 / e32.sum(axis=1, keepdims=True)).astype(jnp.bfloat16)
            ctx32 = lax.dot_general(
                w32, v_ref[:, hcols], (((1,), (0,)), ((), ())),
                preferred_element_type=jnp.float32)
            ctx_ref[0:GLOBAL_TOKENS, hcols] = ctx32.astype(jnp.bfloat16)

    proj = lax.dot_general(
        ctx_ref[...], wo_ref[...], (((1,), (0,)), ((), ())),
        preferred_element_type=jnp.float32)

    def rs_copy(ht, block_r0):
        return pltpu.make_async_remote_copy(
            src_ref=partial_ref.at[pl.ds(block_r0, HBLK)],
            dst_ref=rs_recv_ref.at[ht],
            send_sem=rs_send_sems.at[ht],
            recv_sem=rs_recv_sems.at[ht],
            device_id=(right,),
            device_id_type=pl.DeviceIdType.MESH,
        )

    def ag_copy(c, sem_idx, target, half_sel=None):
        if half_sel is None:
            src = partial_ref.at[_chunk_rows(c)]
        else:
            src = partial_ref.at[pl.ds(c * CHUNK + half_sel * HBLK, HBLK)]
        return pltpu.make_async_remote_copy(
            src_ref=src,
            dst_ref=src,
            send_sem=ag_send_sems.at[sem_idx],
            recv_sem=ag_recv_sems.at[sem_idx],
            device_id=(target,),
            device_id_type=pl.DeviceIdType.MESH,
        )

    if not _COMM:
        partial_ref[rows, :] = proj.astype(jnp.bfloat16)

        @pl.when(t == N_T - 1)
        def _():
            out_ref[...] = partial_ref[...].astype(jnp.float32)
        return

    @pl.when(t == 0)
    def _():
        partial_ref[rows, :] = proj.astype(jnp.bfloat16)
        barrier_sem = pltpu.get_barrier_semaphore()
        for nbr in (left, right):
            pl.semaphore_signal(barrier_sem, inc=1, device_id=(nbr,),
                                device_id_type=pl.DeviceIdType.MESH)
        pl.semaphore_wait(barrier_sem, 2)

    @pl.when(t == 1)
    def _():
        partial_ref[rows, :] = proj.astype(jnp.bfloat16)

    @pl.when(t >= 2)
    def _():
        ht = t - 2
        rs_copy(ht, r0).wait_recv()
        partial_ref[rows, :] = (proj + rs_recv_ref[ht]).astype(jnp.bfloat16)
        rs_copy(ht, r0).wait_send()

    @pl.when(t < N_T - 2)
    def _():
        rs_copy(t, r0).start()

    c_own = lax.rem(my + 1, N_DEV)

    @pl.when(t == N_T - 2)
    def _():
        ag_copy(c_own, 0, right, half_sel=0).start()
        ag_copy(c_own, 1, left, half_sel=0).start()

    @pl.when(t == N_T - 1)
    def _():
        orows = _chunk_rows(c_own)

        ag_copy(c_own, 2, right, half_sel=1).start()
        ag_copy(c_own, 3, left, half_sel=1).start()

        out_ref[orows, :] = partial_ref[orows, :].astype(jnp.float32)

        ag_copy(my, 0, right, half_sel=0).wait_recv()
        fwd_a = ag_copy(my, 4, right, half_sel=0)
        fwd_a.start()

        c_r = lax.rem(my + 2, N_DEV)
        ag_copy(c_r, 1, left, half_sel=0).wait_recv()
        ag_copy(c_r, 3, left, half_sel=1).wait_recv()
        fwd_b = ag_copy(c_r, 5, left, half_sel=1)
        fwd_b.start()
        out_ref[_chunk_rows(c_r), :] = partial_ref[_chunk_rows(c_r), :].astype(
            jnp.float32)

        ag_copy(my, 2, right, half_sel=1).wait_recv()
        out_ref[_chunk_rows(my), :] = partial_ref[_chunk_rows(my), :].astype(
            jnp.float32)

        ag_copy(left, 4, right, half_sel=0).wait_recv()
        ag_copy(left, 5, left, half_sel=1).wait_recv()
        out_ref[_chunk_rows(left), :] = partial_ref[_chunk_rows(left), :].astype(
            jnp.float32)

        ag_copy(c_own, 0, right, half_sel=0).wait_send()
        ag_copy(c_own, 1, left, half_sel=0).wait_send()
        ag_copy(c_own, 2, right, half_sel=1).wait_send()
        ag_copy(c_own, 3, left, half_sel=1).wait_send()
        fwd_a.wait_send()
        fwd_b.wait_send()

        @functools.partial(pl.run_scoped,
                           second_barrier=pltpu.SemaphoreType.REGULAR)
        def _(second_barrier):
            for nbr in (left, right):
                pl.semaphore_signal(second_barrier, inc=1, device_id=(nbr,),
                                    device_id_type=pl.DeviceIdType.MESH)
            pl.semaphore_wait(second_barrier, 2)


def kernel(x, Wq, K_ext, V_ext, Wo):
    my = lax.axis_index("i")
    x2 = x.reshape(SQ, DMODEL).astype(jnp.bfloat16)
    wq_s = lax.dynamic_slice(
        Wq, (0, my * HQ_LOCAL * DH), (DMODEL, HQ_LOCAL * DH)
    ).astype(jnp.bfloat16)
    wo_s = lax.dynamic_slice(
        Wo, (my * HQ_LOCAL * DH, 0), (HQ_LOCAL * DH, DMODEL)
    ).astype(jnp.bfloat16)
    k = K_ext.reshape(SKV, HQ_LOCAL * DH).astype(jnp.bfloat16)
    v = V_ext.reshape(SKV, HQ_LOCAL * DH).astype(jnp.bfloat16)

    out = pl.pallas_call(
        _body,
        grid=(N_T,),
        in_specs=[
            pl.BlockSpec((SQ, DMODEL), lambda t: (0, 0)),
            pl.BlockSpec((DMODEL, HQ_LOCAL * DH), lambda t: (0, 0)),
            pl.BlockSpec((SKV, HQ_LOCAL * DH), lambda t: (0, 0)),
            pl.BlockSpec((SKV, HQ_LOCAL * DH), lambda t: (0, 0)),
            pl.BlockSpec((HQ_LOCAL * DH, DMODEL), lambda t: (0, 0)),
        ],
        out_specs=pl.BlockSpec((SQ, DMODEL), lambda t: (0, 0)),
        out_shape=jax.ShapeDtypeStruct((SQ, DMODEL), jnp.float32),
        scratch_shapes=[
            pltpu.VMEM((SQ, DMODEL), jnp.bfloat16),
            pltpu.VMEM((HBLK, HQ_LOCAL * DH), jnp.bfloat16),
            pltpu.VMEM((N_T - 2, HBLK, DMODEL), jnp.bfloat16),
            pltpu.SemaphoreType.DMA((N_T - 2,)),
            pltpu.SemaphoreType.DMA((N_T - 2,)),
            pltpu.SemaphoreType.DMA((6,)),
            pltpu.SemaphoreType.DMA((6,)),
        ],
        compiler_params=pltpu.CompilerParams(
            dimension_semantics=("arbitrary",),
            **({"collective_id": 0} if _COMM else {}),
        ),
    )(x2, wq_s, k, v, wo_s)
    return out.reshape(1, SQ, DMODEL)


# device time: 90544 ns/iter; 1.8115x vs baseline; 1.0349x over previous
import functools
import os

import jax
import jax.numpy as jnp
from jax import lax
from jax.experimental import pallas as pl
from jax.experimental.pallas import tpu as pltpu

_COMM = os.environ.get("KERNEL_NO_COMM", "0") != "1"
_exp = (lambda s: s + 1.0) if os.environ.get("KERNEL_NO_EXP") == "1" else jnp.exp

N_DEV = 4
SQ = 2048
SKV = 2048
HQ_LOCAL = 8
DH = 128
DMODEL = 1024
N_QB = 4
CHUNK = SQ // N_DEV
HBLK = CHUNK // 2
N_T = 2 * N_QB
SCALE = 0.08838834764831843
LOCAL_WINDOW = 128
GLOBAL_TOKENS = 32
GBLK = 32
WWIN = 640


def _chunk_rows(c):
    return pl.ds(c * CHUNK, CHUNK)


def _body(x_ref, wq_ref, k_ref, v_ref, wo_ref, out_ref,
          partial_ref, ctx_ref, rs_recv_ref, kg_bd, vg_bd, ones_bd,
          rs_send_sems, rs_recv_sems, ag_send_sems, ag_recv_sems):
    t = pl.program_id(0)
    my = lax.axis_index("i")
    left = lax.rem(my + N_DEV - 1, N_DEV)
    right = lax.rem(my + 1, N_DEV)
    hop = lax.div(t, 2)
    half = lax.rem(t, 2)
    qb = lax.rem(my - hop + 2 * N_DEV, N_DEV)
    r0 = qb * CHUNK + half * HBLK
    rows = pl.ds(r0, HBLK)

    @pl.when(t == 0)
    def _():
        kg_bd[...] = jnp.zeros((HQ_LOCAL * GBLK, HQ_LOCAL * DH), jnp.bfloat16)
        vg_bd[...] = jnp.zeros((HQ_LOCAL * GBLK, HQ_LOCAL * DH), jnp.bfloat16)
        for h in range(HQ_LOCAL):
            hcols = slice(h * DH, (h + 1) * DH)
            grows = slice(h * GBLK, (h + 1) * GBLK)
            kg_bd[grows, hcols] = k_ref[0:GBLK, hcols]
            vg_bd[grows, hcols] = v_ref[0:GBLK, hcols]
        ones_bd[...] = (
            lax.broadcasted_iota(jnp.int32, (HQ_LOCAL * GBLK, HQ_LOCAL), 0)
            // GBLK
            == lax.broadcasted_iota(jnp.int32, (HQ_LOCAL * GBLK, HQ_LOCAL), 1)
        ).astype(jnp.bfloat16)

    q_all = (lax.dot_general(
        x_ref[rows, :], wq_ref[...],
        (((1,), (0,)), ((), ())), preferred_element_type=jnp.float32
    ) * SCALE).astype(jnp.bfloat16)

    w0 = jnp.clip(r0 - LOCAL_WINDOW, 0, SKV - WWIN)
    w0 = pl.multiple_of(w0, 128)
    qi_w = r0 + lax.broadcasted_iota(jnp.int32, (HBLK, WWIN), 0)
    ki_w = w0 + lax.broadcasted_iota(jnp.int32, (HBLK, WWIN), 1)
    bias_w = jnp.where(
        (jnp.abs(qi_w - ki_w) <= LOCAL_WINDOW) | (ki_w < GLOBAL_TOKENS),
        0.0, -1e9).astype(jnp.float32)
    bias_g = jnp.where(w0 == 0, -1e9, 0.0).astype(jnp.float32)

    e_g_bf = _exp(lax.dot_general(
        q_all, kg_bd[...], (((1,), (1,)), ((), ())),
        preferred_element_type=jnp.float32) + bias_g).astype(jnp.bfloat16)
    sum_g = lax.dot_general(
        e_g_bf, ones_bd[...], (((1,), (0,)), ((), ())),
        preferred_element_type=jnp.float32)
    ctx_g = lax.dot_general(
        e_g_bf, vg_bd[...], (((1,), (0,)), ((), ())),
        preferred_element_type=jnp.float32)

    for h in range(HQ_LOCAL):
        hcols = slice(h * DH, (h + 1) * DH)
        qh = q_all[:, hcols]
        s_w = lax.dot_general(
            qh, k_ref[pl.ds(w0, WWIN), hcols], (((1,), (1,)), ((), ())),
            preferred_element_type=jnp.float32) + bias_w
        e_w = _exp(s_w)
        inv = 1.0 / (sum_g[:, h:h + 1] + e_w.sum(axis=1, keepdims=True))
        ctx = lax.dot_general(
            e_w.astype(jnp.bfloat16), v_ref[pl.ds(w0, WWIN), hcols],
            (((1,), (0,)), ((), ())), preferred_element_type=jnp.float32)
        ctx_ref[:, hcols] = ((ctx_g[:, hcols] + ctx) * inv).astype(jnp.bfloat16)

    @pl.when(r0 == 0)
    def _():
        for h in range(HQ_LOCAL):
            hcols = slice(h * DH, (h + 1) * DH)
            q32 = q_all[0:GLOBAL_TOKENS, hcols]
            s32 = lax.dot_general(
                q32, k_ref[:, hcols], (((1,), (1,)), ((), ())),
                preferred_element_type=jnp.float32)
            e32 = _exp(s32)
            w32 = (e32 / e32.sum(axis=1, keepdims=True)).astype(jnp.bfloat16)
            ctx32 = lax.dot_general(
                w32, v_ref[:, hcols], (((1,), (0,)), ((), ())),
                preferred_element_type=jnp.float32)
            ctx_ref[0:GLOBAL_TOKENS, hcols] = ctx32.astype(jnp.bfloat16)

    proj = lax.dot_general(
        ctx_ref[...], wo_ref[...], (((1,), (0,)), ((), ())),
        preferred_element_type=jnp.float32)

    def rs_copy(ht, block_r0):
        return pltpu.make_async_remote_copy(
            src_ref=partial_ref.at[pl.ds(block_r0, HBLK)],
            dst_ref=rs_recv_ref.at[ht],
            send_sem=rs_send_sems.at[ht],
            recv_sem=rs_recv_sems.at[ht],
            device_id=(right,),
            device_id_type=pl.DeviceIdType.MESH,
        )

    def ag_copy(c, sem_idx, target, half_sel=None):
        if half_sel is None:
            src = partial_ref.at[_chunk_rows(c)]
        else:
            src = partial_ref.at[pl.ds(c * CHUNK + half_sel * HBLK, HBLK)]
        return pltpu.make_async_remote_copy(
            src_ref=src,
            dst_ref=src,
            send_sem=ag_send_sems.at[sem_idx],
            recv_sem=ag_recv_sems.at[sem_idx],
            device_id=(target,),
            device_id_type=pl.DeviceIdType.MESH,
        )

    if not _COMM:
        partial_ref[rows, :] = proj.astype(jnp.bfloat16)

        @pl.when(t == N_T - 1)
        def _():
            out_ref[...] = partial_ref[...].astype(jnp.float32)
        return

    @pl.when(t == 0)
    def _():
        partial_ref[rows, :] = proj.astype(jnp.bfloat16)
        barrier_sem = pltpu.get_barrier_semaphore()
        for nbr in (left, right):
            pl.semaphore_signal(barrier_sem, inc=1, device_id=(nbr,),
                                device_id_type=pl.DeviceIdType.MESH)
        pl.semaphore_wait(barrier_sem, 2)

    @pl.when(t == 1)
    def _():
        partial_ref[rows, :] = proj.astype(jnp.bfloat16)

    @pl.when(t >= 2)
    def _():
        ht = t - 2
        rs_copy(ht, r0).wait_recv()
        partial_ref[rows, :] = (proj + rs_recv_ref[ht]).astype(jnp.bfloat16)
        rs_copy(ht, r0).wait_send()

    @pl.when(t < N_T - 2)
    def _():
        rs_copy(t, r0).start()

    c_own = lax.rem(my + 1, N_DEV)

    @pl.when(t == N_T - 2)
    def _():
        ag_copy(c_own, 0, right, half_sel=0).start()
        ag_copy(c_own, 1, left, half_sel=0).start()

    @pl.when(t == N_T - 1)
    def _():
        orows = _chunk_rows(c_own)

        ag_copy(c_own, 2, right, half_sel=1).start()
        ag_copy(c_own, 3, left, half_sel=1).start()

        out_ref[orows, :] = partial_ref[orows, :].astype(jnp.float32)

        ag_copy(my, 0, right, half_sel=0).wait_recv()
        fwd_a = ag_copy(my, 4, right, half_sel=0)
        fwd_a.start()

        c_r = lax.rem(my + 2, N_DEV)
        ag_copy(c_r, 1, left, half_sel=0).wait_recv()
        ag_copy(c_r, 3, left, half_sel=1).wait_recv()
        fwd_b = ag_copy(c_r, 5, left, half_sel=1)
        fwd_b.start()
        out_ref[_chunk_rows(c_r), :] = partial_ref[_chunk_rows(c_r), :].astype(
            jnp.float32)

        ag_copy(my, 2, right, half_sel=1).wait_recv()
        out_ref[_chunk_rows(my), :] = partial_ref[_chunk_rows(my), :].astype(
            jnp.float32)

        ag_copy(left, 4, right, half_sel=0).wait_recv()
        ag_copy(left, 5, left, half_sel=1).wait_recv()
        out_ref[_chunk_rows(left), :] = partial_ref[_chunk_rows(left), :].astype(
            jnp.float32)

        ag_copy(c_own, 0, right, half_sel=0).wait_send()
        ag_copy(c_own, 1, left, half_sel=0).wait_send()
        ag_copy(c_own, 2, right, half_sel=1).wait_send()
        ag_copy(c_own, 3, left, half_sel=1).wait_send()
        fwd_a.wait_send()
        fwd_b.wait_send()

        @functools.partial(pl.run_scoped,
                           second_barrier=pltpu.SemaphoreType.REGULAR)
        def _(second_barrier):
            for nbr in (left, right):
                pl.semaphore_signal(second_barrier, inc=1, device_id=(nbr,),
                                    device_id_type=pl.DeviceIdType.MESH)
            pl.semaphore_wait(second_barrier, 2)


def kernel(x, Wq, K_ext, V_ext, Wo):
    my = lax.axis_index("i")
    x2 = x.reshape(SQ, DMODEL).astype(jnp.bfloat16)
    wq_s = lax.dynamic_slice(
        Wq, (0, my * HQ_LOCAL * DH), (DMODEL, HQ_LOCAL * DH)
    ).astype(jnp.bfloat16)
    wo_s = lax.dynamic_slice(
        Wo, (my * HQ_LOCAL * DH, 0), (HQ_LOCAL * DH, DMODEL)
    ).astype(jnp.bfloat16)
    k = K_ext.reshape(SKV, HQ_LOCAL * DH).astype(jnp.bfloat16)
    v = V_ext.reshape(SKV, HQ_LOCAL * DH).astype(jnp.bfloat16)

    out = pl.pallas_call(
        _body,
        grid=(N_T,),
        in_specs=[
            pl.BlockSpec((SQ, DMODEL), lambda t: (0, 0)),
            pl.BlockSpec((DMODEL, HQ_LOCAL * DH), lambda t: (0, 0)),
            pl.BlockSpec((SKV, HQ_LOCAL * DH), lambda t: (0, 0)),
            pl.BlockSpec((SKV, HQ_LOCAL * DH), lambda t: (0, 0)),
            pl.BlockSpec((HQ_LOCAL * DH, DMODEL), lambda t: (0, 0)),
        ],
        out_specs=pl.BlockSpec((SQ, DMODEL), lambda t: (0, 0)),
        out_shape=jax.ShapeDtypeStruct((SQ, DMODEL), jnp.float32),
        scratch_shapes=[
            pltpu.VMEM((SQ, DMODEL), jnp.bfloat16),
            pltpu.VMEM((HBLK, HQ_LOCAL * DH), jnp.bfloat16),
            pltpu.VMEM((N_T - 2, HBLK, DMODEL), jnp.bfloat16),
            pltpu.VMEM((HQ_LOCAL * GBLK, HQ_LOCAL * DH), jnp.bfloat16),
            pltpu.VMEM((HQ_LOCAL * GBLK, HQ_LOCAL * DH), jnp.bfloat16),
            pltpu.VMEM((HQ_LOCAL * GBLK, HQ_LOCAL), jnp.bfloat16),
            pltpu.SemaphoreType.DMA((N_T - 2,)),
            pltpu.SemaphoreType.DMA((N_T - 2,)),
            pltpu.SemaphoreType.DMA((6,)),
            pltpu.SemaphoreType.DMA((6,)),
        ],
        compiler_params=pltpu.CompilerParams(
            dimension_semantics=("arbitrary",),
            **({"collective_id": 0} if _COMM else {}),
        ),
    )(x2, wq_s, k, v, wo_s)
    return out.reshape(1, SQ, DMODEL)
